# baseline (speedup 1.0000x reference)
import functools

import jax
import jax.numpy as jnp
from jax import lax
from jax.experimental import pallas as pl
from jax.experimental.pallas import tpu as pltpu
from jax.experimental.pallas import tpu_sc as plsc


def _make_gather(batch, dim, nc, ns):
    nw = nc * ns
    bpw = batch // nw
    chunk = 128
    nchunk = bpw // chunk
    mesh = plsc.VectorSubcoreMesh(core_axis_name="c", subcore_axis_name="s")

    @functools.partial(
        pl.kernel,
        out_type=[
            jax.ShapeDtypeStruct((batch, dim), jnp.float32),
            jax.ShapeDtypeStruct((batch, dim), jnp.float32),
        ],
        mesh=mesh,
        compiler_params=pltpu.CompilerParams(use_tc_tiling_on_sc=False),
        scratch_types=[
            pltpu.VMEM((nchunk, chunk), jnp.int32),
            pltpu.VMEM((nchunk, chunk), jnp.int32),
            pltpu.VMEM((bpw, dim), jnp.float32),
            pltpu.VMEM((bpw, dim), jnp.float32),
            pltpu.SemaphoreType.DMA,
        ],
    )
    def gather(ut_hbm, it_hbm, uid_hbm, iid_hbm, ue_hbm, ie_hbm,
               uidx_v, iidx_v, urows_v, irows_v, sem):
        wid = lax.axis_index("s") * nc + lax.axis_index("c")
        base = wid * bpw
        pltpu.sync_copy(uid_hbm.at[pl.ds(wid * nchunk, nchunk)], uidx_v)
        pltpu.sync_copy(iid_hbm.at[pl.ds(wid * nchunk, nchunk)], iidx_v)
        copies = []
        for k in range(nchunk):
            copies.append(pltpu.async_copy(
                ut_hbm.at[uidx_v.at[k]], urows_v.at[pl.ds(k * chunk, chunk)], sem))
            copies.append(pltpu.async_copy(
                it_hbm.at[iidx_v.at[k]], irows_v.at[pl.ds(k * chunk, chunk)], sem))
        for c in copies:
            c.wait()
        pltpu.sync_copy(urows_v, ue_hbm.at[pl.ds(base, bpw)])
        pltpu.sync_copy(irows_v, ie_hbm.at[pl.ds(base, bpw)])

    return gather


def _mlp_body(ue_ref, ie_ref, uf_ref, if_ref, w1u_ref, w1i_ref, w1f_ref,
              b1_ref, w2_ref, b2_ref, out_ref):
    h = jnp.dot(ue_ref[...], w1u_ref[...], preferred_element_type=jnp.float32)
    h += jnp.dot(ie_ref[...], w1i_ref[...], preferred_element_type=jnp.float32)
    h += uf_ref[...] * w1f_ref[0:1, :]
    h += if_ref[...] * w1f_ref[1:2, :]
    h = jnp.maximum(h + b1_ref[...], 0.0)
    y = jnp.dot(h, w2_ref[...], preferred_element_type=jnp.float32) + b2_ref[...]
    out_ref[...] = jax.nn.sigmoid(y)


def _make_mlp(batch, dim, hidden, blk):
    grid = (batch // blk,)
    row = lambda i: (i, 0)
    fixed = lambda i: (0, 0)
    return pl.pallas_call(
        _mlp_body,
        grid=grid,
        in_specs=[
            pl.BlockSpec((blk, dim), row),
            pl.BlockSpec((blk, dim), row),
            pl.BlockSpec((blk, 1), row),
            pl.BlockSpec((blk, 1), row),
            pl.BlockSpec((dim, hidden), fixed),
            pl.BlockSpec((dim, hidden), fixed),
            pl.BlockSpec((8, hidden), fixed),
            pl.BlockSpec((1, hidden), fixed),
            pl.BlockSpec((hidden, 1), fixed),
            pl.BlockSpec((1, 1), fixed),
        ],
        out_specs=pl.BlockSpec((blk, 1), row),
        out_shape=jax.ShapeDtypeStruct((batch, 1), jnp.float32),
    )


def kernel(user_id, item_id, user_feature, item_feature, user_table,
           item_table, W1, b1, W2, b2):
    batch = user_id.shape[0]
    dim = user_table.shape[1]
    hidden = W1.shape[1]
    info = plsc.get_sparse_core_info()
    nc, ns = info.num_cores, info.num_subcores

    gather = _make_gather(batch, dim, nc, ns)
    ue, ie = gather(user_table, item_table,
                    user_id.reshape(batch // 128, 128),
                    item_id.reshape(batch // 128, 128))

    w1f = jnp.concatenate(
        [W1[2 * dim:], jnp.zeros((8 - (W1.shape[0] - 2 * dim), hidden), W1.dtype)], axis=0)
    mlp = _make_mlp(batch, dim, hidden, 1024)
    y = mlp(ue, ie, user_feature.reshape(batch, 1),
            item_feature.reshape(batch, 1), W1[:dim], W1[dim:2 * dim],
            w1f, b1.reshape(1, hidden), W2, b2.reshape(1, 1))
    return y.reshape(batch)
